# baseline (speedup 1.0000x reference)
import jax
import jax.numpy as jnp
from jax import lax
from jax.experimental import pallas as pl
from jax.experimental.pallas import tpu as pltpu
from jax.experimental.pallas import tpu_sc as plsc

N = 10000
E = 320000
DF = 128
DI = 32
D0 = DF + DI
DH = 224
HALF = DH // 2
NG = 64
NC = 2
NS = 16
TAB = 10240
ROWS_PT = TAB // NS
LAST_PT = N - (NS - 1) * ROWS_PT
W = 80
NROW = E // W
DEPTH = 2
DEPTH1 = 3
RB = 1000
GRID = N // RB
RBC = 1000
GRIDC = N // RBC
F32 = jnp.float32

_sc_mesh = plsc.VectorSubcoreMesh(core_axis_name="c", subcore_axis_name="s")


def _t0_body(x_ref, b_ref, s_ref, o_ref):
    oh = (b_ref[...] == lax.broadcasted_iota(jnp.int32, (1, NG), 1)).astype(F32)
    rep = jnp.dot(oh, s_ref[...], preferred_element_type=F32)
    o_ref[...] = jnp.concatenate([x_ref[...], rep], axis=1)


_t0_call = pl.pallas_call(
    _t0_body,
    grid=(GRID,),
    in_specs=[
        pl.BlockSpec((RB, DF), lambda i: (i, 0)),
        pl.BlockSpec((RB, 1), lambda i: (i, 0)),
        pl.BlockSpec((NG, DI), lambda i: (0, 0)),
    ],
    out_specs=pl.BlockSpec((RB, D0), lambda i: (i, 0)),
    out_shape=jax.ShapeDtypeStruct((N, D0), F32),
)


def _sc_deg_body(dst_hbm, deg_hbm, deg_sh, ones_v, zd_v, idst_v, sem):
    c = lax.axis_index("c")
    s = lax.axis_index("s")
    wid = c * NS + s
    zeros16 = jnp.zeros((16,), F32)
    ones16 = jnp.ones((16,), F32)

    @pl.loop(0, W)
    def _(i):
        ones_v[i] = ones16
        zd_v[i] = zeros16

    rbase = s * ROWS_PT
    @pl.loop(0, ROWS_PT // W)
    def _(k):
        pltpu.sync_copy(zd_v, deg_sh.at[pl.ds(rbase + k * W, W)])

    nch = NROW // (NC * NS)
    pltpu.sync_copy(dst_hbm.at[pl.ds(wid * nch, nch)], idst_v)
    plsc.subcore_barrier()

    @pl.loop(0, nch, step=5)
    def _(ch):
        for k in range(5):
            pltpu.async_copy(ones_v, deg_sh.at[idst_v.at[ch + k]], sem,
                             add=True)
        for k in range(5):
            pltpu.make_async_copy(ones_v, deg_sh.at[idst_v.at[ch + k]],
                                  sem).wait()

    plsc.subcore_barrier()

    obase = c * N + rbase

    @pl.when(s < NS - 1)
    def _():
        pltpu.sync_copy(deg_sh.at[pl.ds(rbase, ROWS_PT)],
                        deg_hbm.at[pl.ds(obase, ROWS_PT)])

    @pl.when(s == NS - 1)
    def _():
        pltpu.sync_copy(deg_sh.at[pl.ds(rbase, LAST_PT)],
                        deg_hbm.at[pl.ds(obase, LAST_PT)])


_deg_call = pl.kernel(
    _sc_deg_body,
    out_type=jax.ShapeDtypeStruct((NC * N, 16), F32),
    mesh=_sc_mesh,
    scratch_types=[
        pltpu.VMEM_SHARED((TAB, 16), F32),
        pltpu.VMEM((W, 16), F32),
        pltpu.VMEM((W, 16), F32),
        pltpu.VMEM((NROW // (NC * NS), W), jnp.int32),
        pltpu.SemaphoreType.DMA,
    ],
    compiler_params=pltpu.CompilerParams(use_tc_tiling_on_sc=False),
)


def _edge_pipeline(tbl_hbm, src_hbm, dst_hbm, tab_sh,
                   isrc_v, idst_v, rows_bufs, sems, semi,
                   tile_row_base, n_chunks):
    depth = len(rows_bufs)
    n_blocks = n_chunks // depth
    tail = n_chunks % depth

    def start(idx_row, rows_v, sem):
        return pltpu.async_copy(tbl_hbm.at[isrc_v.at[idx_row]], rows_v, sem)

    def wait(idx_row, rows_v, sem):
        pltpu.make_async_copy(tbl_hbm.at[isrc_v.at[idx_row]], rows_v,
                              sem).wait()

    def scatter(idx_row, rows_v):
        pltpu.sync_copy(rows_v, tab_sh.at[idst_v.at[idx_row]], add=True)

    def idx_load(blk, half, sync, rows=depth):
        nb = tile_row_base + blk * depth
        if sync:
            pltpu.sync_copy(src_hbm.at[pl.ds(nb, rows)],
                            isrc_v.at[pl.ds(half, rows)])
            pltpu.sync_copy(dst_hbm.at[pl.ds(nb, rows)],
                            idst_v.at[pl.ds(half, rows)])
        else:
            pltpu.async_copy(src_hbm.at[pl.ds(nb, rows)],
                             isrc_v.at[pl.ds(half, rows)], semi)
            pltpu.async_copy(dst_hbm.at[pl.ds(nb, rows)],
                             idst_v.at[pl.ds(half, rows)], semi)

    def idx_wait(blk, half):
        nb = tile_row_base + blk * depth
        pltpu.make_async_copy(src_hbm.at[pl.ds(nb, depth)],
                              isrc_v.at[pl.ds(half, depth)], semi).wait()
        pltpu.make_async_copy(dst_hbm.at[pl.ds(nb, depth)],
                              idst_v.at[pl.ds(half, depth)], semi).wait()

    idx_load(0, 0, True)
    start(0, rows_bufs[0], sems[0])
    if n_blocks > 1:
        idx_load(1, depth, False)

    @pl.loop(0, n_blocks)
    def _(blk):
        half = lax.rem(blk, 2) * depth
        nhalf = depth - half
        for j in range(depth - 1):
            start(half + j + 1, rows_bufs[j + 1], sems[j + 1])
        wait(half, rows_bufs[0], sems[0])
        scatter(half, rows_bufs[0])

        @pl.when(blk < n_blocks - 1)
        def _():
            idx_wait(blk + 1, nhalf)
            start(nhalf, rows_bufs[0], sems[0])

        for j in range(1, depth):
            wait(half + j, rows_bufs[j], sems[j])
            scatter(half + j, rows_bufs[j])

        @pl.when(blk < n_blocks - 2)
        def _():
            idx_load(blk + 2, half, False)

    if tail:
        last = tile_row_base + n_blocks * depth
        pltpu.sync_copy(src_hbm.at[pl.ds(last, tail)],
                        isrc_v.at[pl.ds(0, tail)])
        pltpu.sync_copy(dst_hbm.at[pl.ds(last, tail)],
                        idst_v.at[pl.ds(0, tail)])
        for j in range(tail):
            start(j, rows_bufs[0], sems[0]).wait()
            scatter(j, rows_bufs[0])


def _zero_fill(rows_v, tab_sh, s, width):
    zeros16 = jnp.zeros((16,), F32)

    @pl.loop(0, W)
    def _(i):
        @pl.loop(0, width // 16)
        def _(j):
            rows_v[i, pl.ds(j * 16, 16)] = zeros16

    rbase = s * ROWS_PT
    @pl.loop(0, ROWS_PT // W)
    def _(k):
        pltpu.sync_copy(rows_v, tab_sh.at[pl.ds(rbase + k * W, W)])


def _write_out(tab_sh, out_hbm, c, s):
    rbase = s * ROWS_PT
    obase = c * N + rbase

    @pl.when(s < NS - 1)
    def _():
        pltpu.sync_copy(tab_sh.at[pl.ds(rbase, ROWS_PT)],
                        out_hbm.at[pl.ds(obase, ROWS_PT)])

    @pl.when(s == NS - 1)
    def _():
        pltpu.sync_copy(tab_sh.at[pl.ds(rbase, LAST_PT)],
                        out_hbm.at[pl.ds(obase, LAST_PT)])


def _sc_l0_body(t0_hbm, src_hbm, dst_hbm, agg_hbm,
                tab_sh, isrc_v, idst_v, *bufs_sems):
    bufs = list(bufs_sems[:DEPTH])
    sems = list(bufs_sems[DEPTH:2 * DEPTH])
    semi = bufs_sems[2 * DEPTH]
    c = lax.axis_index("c")
    s = lax.axis_index("s")
    wid = c * NS + s
    _zero_fill(bufs[0], tab_sh, s, D0)
    plsc.subcore_barrier()
    _edge_pipeline(t0_hbm, src_hbm, dst_hbm, tab_sh,
                   isrc_v, idst_v, bufs, sems,
                   semi, wid * (NROW // (NC * NS)), NROW // (NC * NS))
    plsc.subcore_barrier()
    _write_out(tab_sh, agg_hbm, c, s)


_l0_call = pl.kernel(
    _sc_l0_body,
    out_type=jax.ShapeDtypeStruct((NC * N, D0), F32),
    mesh=_sc_mesh,
    scratch_types=(
        [pltpu.VMEM_SHARED((TAB, D0), F32),
         pltpu.VMEM((2 * DEPTH, W), jnp.int32),
         pltpu.VMEM((2 * DEPTH, W), jnp.int32)]
        + [pltpu.VMEM((W, D0), F32)] * DEPTH
        + [pltpu.SemaphoreType.DMA] * (DEPTH + 1)
    ),
    compiler_params=pltpu.CompilerParams(use_tc_tiling_on_sc=False),
)


def _mid_body(t0_ref, aggA_ref, aggB_ref, degA_ref, degB_ref,
              wl0_ref, bl0_ref, wr0_ref, wl1_ref, wr1_ref,
              h1_ref, p1_ref, r1_ref):
    deg = jnp.maximum(degA_ref[:, 0:1] + degB_ref[:, 0:1], 1.0)
    mean = (aggA_ref[...] + aggB_ref[...]) / deg
    t0 = t0_ref[...]
    h1 = (jnp.dot(mean, wl0_ref[...], preferred_element_type=F32)
          + bl0_ref[...]
          + jnp.dot(t0, wr0_ref[...], preferred_element_type=F32))
    h1_ref[...] = h1
    rep = t0[:, DF:]
    p1 = (jnp.dot(h1, wl1_ref[0:DH, :], preferred_element_type=F32)
          + jnp.dot(rep, wl1_ref[DH:, :], preferred_element_type=F32))
    p1_ref[0] = p1[:, :HALF]
    p1_ref[1] = p1[:, HALF:]
    r1_ref[...] = (jnp.dot(h1, wr1_ref[0:DH, :], preferred_element_type=F32)
                   + jnp.dot(rep, wr1_ref[DH:, :], preferred_element_type=F32))


_mid_call = pl.pallas_call(
    _mid_body,
    grid=(GRID,),
    in_specs=[
        pl.BlockSpec((RB, D0), lambda i: (i, 0)),
        pl.BlockSpec((RB, D0), lambda i: (i, 0)),
        pl.BlockSpec((RB, D0), lambda i: (i + GRID, 0)),
        pl.BlockSpec((RB, 16), lambda i: (i, 0)),
        pl.BlockSpec((RB, 16), lambda i: (i + GRID, 0)),
        pl.BlockSpec((D0, DH), lambda i: (0, 0)),
        pl.BlockSpec((1, DH), lambda i: (0, 0)),
        pl.BlockSpec((D0, DH), lambda i: (0, 0)),
        pl.BlockSpec((DH + DI, DH), lambda i: (0, 0)),
        pl.BlockSpec((DH + DI, DH), lambda i: (0, 0)),
    ],
    out_specs=[
        pl.BlockSpec((RB, DH), lambda i: (i, 0)),
        pl.BlockSpec((2, RB, HALF), lambda i: (0, i, 0)),
        pl.BlockSpec((RB, DH), lambda i: (i, 0)),
    ],
    out_shape=[
        jax.ShapeDtypeStruct((N, DH), F32),
        jax.ShapeDtypeStruct((2, N, HALF), F32),
        jax.ShapeDtypeStruct((N, DH), F32),
    ],
)


def _sc_l1_body(p1a_hbm, p1b_hbm, src_hbm, dst_hbm, out_hbm,
                tab_sh, isrc_v, idst_v, *bufs_sems):
    bufs = list(bufs_sems[:DEPTH1])
    sems = list(bufs_sems[DEPTH1:2 * DEPTH1])
    semi = bufs_sems[2 * DEPTH1]
    c = lax.axis_index("c")
    s = lax.axis_index("s")
    _zero_fill(bufs[0], tab_sh, s, HALF)
    plsc.subcore_barrier()

    trb = s * (NROW // NS)
    nch = NROW // NS

    @pl.when(c == 0)
    def _():
        _edge_pipeline(p1a_hbm, src_hbm, dst_hbm, tab_sh,
                       isrc_v, idst_v, bufs, sems, semi, trb, nch)

    @pl.when(c == 1)
    def _():
        _edge_pipeline(p1b_hbm, src_hbm, dst_hbm, tab_sh,
                       isrc_v, idst_v, bufs, sems, semi, trb, nch)

    plsc.subcore_barrier()
    _write_out(tab_sh, out_hbm, c, s)


_l1_call = pl.kernel(
    _sc_l1_body,
    out_type=jax.ShapeDtypeStruct((NC * N, HALF), F32),
    mesh=_sc_mesh,
    scratch_types=(
        [pltpu.VMEM_SHARED((TAB, HALF), F32),
         pltpu.VMEM((2 * DEPTH1, W), jnp.int32),
         pltpu.VMEM((2 * DEPTH1, W), jnp.int32)]
        + [pltpu.VMEM((W, HALF), F32)] * DEPTH1
        + [pltpu.SemaphoreType.DMA] * (DEPTH1 + 1)
    ),
    compiler_params=pltpu.CompilerParams(use_tc_tiling_on_sc=False),
)


def _out_body(bs_ref, batch_ref, h1_ref, a1A_ref, a1B_ref, degA_ref,
              degB_ref, r1_ref, bl1_ref, f1w_ref, f1b_ref, f2w_ref,
              f2b_ref, o_ref, acc_ref, hv_ref):
    i = pl.program_id(0)
    deg = jnp.maximum(degA_ref[:, 0:1] + degB_ref[:, 0:1], 1.0)
    h2 = (jnp.concatenate([a1A_ref[...], a1B_ref[...]], axis=1) / deg
          + bl1_ref[...] + r1_ref[...])
    hv_ref[...] = jnp.concatenate([h1_ref[...], h2], axis=1)

    @pl.when(i == 0)
    def _():
        acc_ref[...] = jnp.full((NG, 2 * DH), -jnp.inf, F32)

    bmin = bs_ref[i * RBC]
    bmax = bs_ref[i * RBC + RBC - 1]
    gi = lax.broadcasted_iota(jnp.int32, (NG, 1), 0)

    def body(g, carry):
        m = jnp.where(batch_ref[...] == g, hv_ref[...], -jnp.inf)
        m = jnp.max(m, axis=0, keepdims=True)
        acc_ref[...] = jnp.where(gi == g, jnp.maximum(acc_ref[...], m),
                                 acc_ref[...])
        return carry

    lax.fori_loop(bmin, bmax + 1, body, 0)

    @pl.when(i == pl.num_programs(0) - 1)
    def _():
        z = jnp.maximum(
            jnp.dot(acc_ref[...], f1w_ref[...], preferred_element_type=F32)
            + f1b_ref[...], 0.0)
        o_ref[...] = (jnp.dot(z, f2w_ref[...], preferred_element_type=F32)
                      + f2b_ref[...])


_out_call = pl.pallas_call(
    _out_body,
    grid_spec=pltpu.PrefetchScalarGridSpec(
        num_scalar_prefetch=1,
        grid=(GRIDC,),
        in_specs=[
            pl.BlockSpec((RBC, 1), lambda i, b: (i, 0)),
            pl.BlockSpec((RBC, DH), lambda i, b: (i, 0)),
            pl.BlockSpec((RBC, HALF), lambda i, b: (i, 0)),
            pl.BlockSpec((RBC, HALF), lambda i, b: (i + GRIDC, 0)),
            pl.BlockSpec((RBC, 16), lambda i, b: (i, 0)),
            pl.BlockSpec((RBC, 16), lambda i, b: (i + GRIDC, 0)),
            pl.BlockSpec((RBC, DH), lambda i, b: (i, 0)),
            pl.BlockSpec((1, DH), lambda i, b: (0, 0)),
            pl.BlockSpec((2 * DH, 256), lambda i, b: (0, 0)),
            pl.BlockSpec((1, 256), lambda i, b: (0, 0)),
            pl.BlockSpec((256, 10), lambda i, b: (0, 0)),
            pl.BlockSpec((1, 10), lambda i, b: (0, 0)),
        ],
        out_specs=pl.BlockSpec((NG, 10), lambda i, b: (0, 0)),
        scratch_shapes=[
            pltpu.VMEM((NG, 2 * DH), F32),
            pltpu.VMEM((RBC, 2 * DH), F32),
        ],
    ),
    out_shape=jax.ShapeDtypeStruct((NG, 10), F32),
)


@jax.jit
def _run(x, edge_index, batch, eigen_values, W_l0, b_l0, W_r0,
         W_l1, b_l1, W_r1, fc1_w, fc1_b, fc2_w, fc2_b):
    srcs = edge_index[0].reshape(NROW, W)
    dsts = edge_index[1].reshape(NROW, W)
    batch2 = batch.reshape(N, 1)
    deg = _deg_call(dsts)
    t0 = _t0_call(x, batch2, eigen_values)
    agg0 = _l0_call(t0, srcs, dsts)
    h1, p1, r1 = _mid_call(t0, agg0, agg0, deg, deg,
                           W_l0, b_l0.reshape(1, DH), W_r0, W_l1, W_r1)
    agg1 = _l1_call(p1[0], p1[1], srcs, dsts)
    return _out_call(batch, batch2, h1, agg1, agg1, deg, deg, r1,
                     b_l1.reshape(1, DH), fc1_w, fc1_b.reshape(1, 256),
                     fc2_w, fc2_b.reshape(1, 10))


def kernel(x, edge_index, batch, eigen_values, W_l0, b_l0, W_r0,
           W_l1, b_l1, W_r1, fc1_w, fc1_b, fc2_w, fc2_b):
    return _run(x, edge_index, batch, eigen_values, W_l0, b_l0, W_r0,
                W_l1, b_l1, W_r1, fc1_w, fc1_b, fc2_w, fc2_b)


# speedup vs baseline: 1.0227x; 1.0227x over previous
import jax
import jax.numpy as jnp
from jax import lax
from jax.experimental import pallas as pl
from jax.experimental.pallas import tpu as pltpu
from jax.experimental.pallas import tpu_sc as plsc

N = 10000
E = 320000
DF = 128
DI = 32
D0 = DF + DI
DH = 224
HALF = DH // 2
NG = 64
NC = 2
NS = 16
TAB = 10240
ROWS_PT = TAB // NS
LAST_PT = N - (NS - 1) * ROWS_PT
W = 80
NROW = E // W
DEPTH = 2
DEPTH1 = 2
RB = 1000
GRID = N // RB
RBC = 1000
GRIDC = N // RBC
F32 = jnp.float32

_sc_mesh = plsc.VectorSubcoreMesh(core_axis_name="c", subcore_axis_name="s")


def _t0_body(x_ref, b_ref, s_ref, o_ref):
    oh = (b_ref[...] == lax.broadcasted_iota(jnp.int32, (1, NG), 1)).astype(F32)
    rep = jnp.dot(oh, s_ref[...], preferred_element_type=F32)
    o_ref[...] = jnp.concatenate([x_ref[...], rep], axis=1)


_t0_call = pl.pallas_call(
    _t0_body,
    grid=(GRID,),
    in_specs=[
        pl.BlockSpec((RB, DF), lambda i: (i, 0)),
        pl.BlockSpec((RB, 1), lambda i: (i, 0)),
        pl.BlockSpec((NG, DI), lambda i: (0, 0)),
    ],
    out_specs=pl.BlockSpec((RB, D0), lambda i: (i, 0)),
    out_shape=jax.ShapeDtypeStruct((N, D0), F32),
)


def _sc_deg_body(edges_hbm, deg_hbm, deg_sh, ones_v, zd_v, idst_v, sem):
    c = lax.axis_index("c")
    s = lax.axis_index("s")
    wid = c * NS + s
    zeros16 = jnp.zeros((16,), F32)
    ones16 = jnp.ones((16,), F32)

    @pl.loop(0, W)
    def _(i):
        ones_v[i] = ones16
        zd_v[i] = zeros16

    rbase = s * ROWS_PT
    @pl.loop(0, ROWS_PT // W)
    def _(k):
        pltpu.sync_copy(zd_v, deg_sh.at[pl.ds(rbase + k * W, W)])

    nch = NROW // (NC * NS)
    pltpu.sync_copy(edges_hbm.at[pl.ds(NROW + wid * nch, nch)], idst_v)
    plsc.subcore_barrier()

    @pl.loop(0, nch, step=5)
    def _(ch):
        for k in range(5):
            pltpu.async_copy(ones_v, deg_sh.at[idst_v.at[ch + k]], sem,
                             add=True)
        for k in range(5):
            pltpu.make_async_copy(ones_v, deg_sh.at[idst_v.at[ch + k]],
                                  sem).wait()

    plsc.subcore_barrier()

    obase = c * N + rbase

    @pl.when(s < NS - 1)
    def _():
        pltpu.sync_copy(deg_sh.at[pl.ds(rbase, ROWS_PT)],
                        deg_hbm.at[pl.ds(obase, ROWS_PT)])

    @pl.when(s == NS - 1)
    def _():
        pltpu.sync_copy(deg_sh.at[pl.ds(rbase, LAST_PT)],
                        deg_hbm.at[pl.ds(obase, LAST_PT)])


_deg_call = pl.kernel(
    _sc_deg_body,
    out_type=jax.ShapeDtypeStruct((NC * N, 16), F32),
    mesh=_sc_mesh,
    scratch_types=[
        pltpu.VMEM_SHARED((TAB, 16), F32),
        pltpu.VMEM((W, 16), F32),
        pltpu.VMEM((W, 16), F32),
        pltpu.VMEM((NROW // (NC * NS), W), jnp.int32),
        pltpu.SemaphoreType.DMA,
    ],
    compiler_params=pltpu.CompilerParams(use_tc_tiling_on_sc=False),
)


def _edge_pipeline(tbl_hbm, edges_hbm, tab_sh,
                   isrc_v, idst_v, rows_bufs, sems, semi,
                   tile_row_base, n_chunks):
    src_hbm = dst_hbm = edges_hbm
    dst_off = NROW
    depth = len(rows_bufs)
    n_blocks = n_chunks // depth
    tail = n_chunks % depth

    def start(idx_row, rows_v, sem):
        return pltpu.async_copy(tbl_hbm.at[isrc_v.at[idx_row]], rows_v, sem)

    def wait(idx_row, rows_v, sem):
        pltpu.make_async_copy(tbl_hbm.at[isrc_v.at[idx_row]], rows_v,
                              sem).wait()

    def scatter(idx_row, rows_v):
        pltpu.sync_copy(rows_v, tab_sh.at[idst_v.at[idx_row]], add=True)

    def idx_load(blk, half, sync, rows=depth):
        nb = tile_row_base + blk * depth
        if sync:
            pltpu.sync_copy(src_hbm.at[pl.ds(nb, rows)],
                            isrc_v.at[pl.ds(half, rows)])
            pltpu.sync_copy(dst_hbm.at[pl.ds(nb + dst_off, rows)],
                            idst_v.at[pl.ds(half, rows)])
        else:
            pltpu.async_copy(src_hbm.at[pl.ds(nb, rows)],
                             isrc_v.at[pl.ds(half, rows)], semi)
            pltpu.async_copy(dst_hbm.at[pl.ds(nb + dst_off, rows)],
                             idst_v.at[pl.ds(half, rows)], semi)

    def idx_wait(blk, half):
        nb = tile_row_base + blk * depth
        pltpu.make_async_copy(src_hbm.at[pl.ds(nb, depth)],
                              isrc_v.at[pl.ds(half, depth)], semi).wait()
        pltpu.make_async_copy(dst_hbm.at[pl.ds(nb + dst_off, depth)],
                              idst_v.at[pl.ds(half, depth)], semi).wait()

    idx_load(0, 0, True)
    start(0, rows_bufs[0], sems[0])
    if n_blocks > 1:
        idx_load(1, depth, False)

    @pl.loop(0, n_blocks)
    def _(blk):
        half = lax.rem(blk, 2) * depth
        nhalf = depth - half
        for j in range(depth - 1):
            start(half + j + 1, rows_bufs[j + 1], sems[j + 1])
        wait(half, rows_bufs[0], sems[0])
        scatter(half, rows_bufs[0])

        @pl.when(blk < n_blocks - 1)
        def _():
            idx_wait(blk + 1, nhalf)
            start(nhalf, rows_bufs[0], sems[0])

        for j in range(1, depth):
            wait(half + j, rows_bufs[j], sems[j])
            scatter(half + j, rows_bufs[j])

        @pl.when(blk < n_blocks - 2)
        def _():
            idx_load(blk + 2, half, False)

    if tail:
        last = tile_row_base + n_blocks * depth
        pltpu.sync_copy(src_hbm.at[pl.ds(last, tail)],
                        isrc_v.at[pl.ds(0, tail)])
        pltpu.sync_copy(dst_hbm.at[pl.ds(last + dst_off, tail)],
                        idst_v.at[pl.ds(0, tail)])
        for j in range(tail):
            start(j, rows_bufs[0], sems[0]).wait()
            scatter(j, rows_bufs[0])


def _zero_fill(rows_v, tab_sh, s, width):
    zeros16 = jnp.zeros((16,), F32)

    @pl.loop(0, W)
    def _(i):
        @pl.loop(0, width // 16)
        def _(j):
            rows_v[i, pl.ds(j * 16, 16)] = zeros16

    rbase = s * ROWS_PT
    @pl.loop(0, ROWS_PT // W)
    def _(k):
        pltpu.sync_copy(rows_v, tab_sh.at[pl.ds(rbase + k * W, W)])


def _write_out(tab_sh, out_hbm, c, s):
    rbase = s * ROWS_PT
    obase = c * N + rbase

    @pl.when(s < NS - 1)
    def _():
        pltpu.sync_copy(tab_sh.at[pl.ds(rbase, ROWS_PT)],
                        out_hbm.at[pl.ds(obase, ROWS_PT)])

    @pl.when(s == NS - 1)
    def _():
        pltpu.sync_copy(tab_sh.at[pl.ds(rbase, LAST_PT)],
                        out_hbm.at[pl.ds(obase, LAST_PT)])


def _sc_l0_body(t0_hbm, edges_hbm, agg_hbm,
                tab_sh, isrc_v, idst_v, *bufs_sems):
    bufs = list(bufs_sems[:DEPTH])
    sems = list(bufs_sems[DEPTH:2 * DEPTH])
    semi = bufs_sems[2 * DEPTH]
    c = lax.axis_index("c")
    s = lax.axis_index("s")
    wid = c * NS + s
    _zero_fill(bufs[0], tab_sh, s, D0)
    plsc.subcore_barrier()
    _edge_pipeline(t0_hbm, edges_hbm, tab_sh,
                   isrc_v, idst_v, bufs, sems,
                   semi, wid * (NROW // (NC * NS)), NROW // (NC * NS))
    plsc.subcore_barrier()
    _write_out(tab_sh, agg_hbm, c, s)


_l0_call = pl.kernel(
    _sc_l0_body,
    out_type=jax.ShapeDtypeStruct((NC * N, D0), F32),
    mesh=_sc_mesh,
    scratch_types=(
        [pltpu.VMEM_SHARED((TAB, D0), F32),
         pltpu.VMEM((2 * DEPTH, W), jnp.int32),
         pltpu.VMEM((2 * DEPTH, W), jnp.int32)]
        + [pltpu.VMEM((W, D0), F32)] * DEPTH
        + [pltpu.SemaphoreType.DMA] * (DEPTH + 1)
    ),
    compiler_params=pltpu.CompilerParams(use_tc_tiling_on_sc=False),
)


def _mid_body(t0_ref, aggA_ref, aggB_ref, degA_ref, degB_ref,
              wl0_ref, bl0_ref, wr0_ref, wl1_ref, wr1_ref,
              h1_ref, p1_ref, r1_ref):
    deg = jnp.maximum(degA_ref[:, 0:1] + degB_ref[:, 0:1], 1.0)
    mean = (aggA_ref[...] + aggB_ref[...]) / deg
    t0 = t0_ref[...]
    h1 = (jnp.dot(mean, wl0_ref[...], preferred_element_type=F32)
          + bl0_ref[...]
          + jnp.dot(t0, wr0_ref[...], preferred_element_type=F32))
    h1_ref[...] = h1
    rep = t0[:, DF:]
    p1 = (jnp.dot(h1, wl1_ref[0:DH, :], preferred_element_type=F32)
          + jnp.dot(rep, wl1_ref[DH:, :], preferred_element_type=F32))
    p1_ref[0] = p1[:, :HALF]
    p1_ref[1] = p1[:, HALF:]
    r1_ref[...] = (jnp.dot(h1, wr1_ref[0:DH, :], preferred_element_type=F32)
                   + jnp.dot(rep, wr1_ref[DH:, :], preferred_element_type=F32))


_mid_call = pl.pallas_call(
    _mid_body,
    grid=(GRID,),
    in_specs=[
        pl.BlockSpec((RB, D0), lambda i: (i, 0)),
        pl.BlockSpec((RB, D0), lambda i: (i, 0)),
        pl.BlockSpec((RB, D0), lambda i: (i + GRID, 0)),
        pl.BlockSpec((RB, 16), lambda i: (i, 0)),
        pl.BlockSpec((RB, 16), lambda i: (i + GRID, 0)),
        pl.BlockSpec((D0, DH), lambda i: (0, 0)),
        pl.BlockSpec((1, DH), lambda i: (0, 0)),
        pl.BlockSpec((D0, DH), lambda i: (0, 0)),
        pl.BlockSpec((DH + DI, DH), lambda i: (0, 0)),
        pl.BlockSpec((DH + DI, DH), lambda i: (0, 0)),
    ],
    out_specs=[
        pl.BlockSpec((RB, DH), lambda i: (i, 0)),
        pl.BlockSpec((2, RB, HALF), lambda i: (0, i, 0)),
        pl.BlockSpec((RB, DH), lambda i: (i, 0)),
    ],
    out_shape=[
        jax.ShapeDtypeStruct((N, DH), F32),
        jax.ShapeDtypeStruct((2, N, HALF), F32),
        jax.ShapeDtypeStruct((N, DH), F32),
    ],
)


def _sc_l1_body(p1a_hbm, p1b_hbm, edges_hbm, out_hbm,
                tab_sh, isrc_v, idst_v, *bufs_sems):
    bufs = list(bufs_sems[:DEPTH1])
    sems = list(bufs_sems[DEPTH1:2 * DEPTH1])
    semi = bufs_sems[2 * DEPTH1]
    c = lax.axis_index("c")
    s = lax.axis_index("s")
    _zero_fill(bufs[0], tab_sh, s, HALF)
    plsc.subcore_barrier()

    trb = s * (NROW // NS)
    nch = NROW // NS

    @pl.when(c == 0)
    def _():
        _edge_pipeline(p1a_hbm, edges_hbm, tab_sh,
                       isrc_v, idst_v, bufs, sems, semi, trb, nch)

    @pl.when(c == 1)
    def _():
        _edge_pipeline(p1b_hbm, edges_hbm, tab_sh,
                       isrc_v, idst_v, bufs, sems, semi, trb, nch)

    plsc.subcore_barrier()
    _write_out(tab_sh, out_hbm, c, s)


_l1_call = pl.kernel(
    _sc_l1_body,
    out_type=jax.ShapeDtypeStruct((NC * N, HALF), F32),
    mesh=_sc_mesh,
    scratch_types=(
        [pltpu.VMEM_SHARED((TAB, HALF), F32),
         pltpu.VMEM((2 * DEPTH1, W), jnp.int32),
         pltpu.VMEM((2 * DEPTH1, W), jnp.int32)]
        + [pltpu.VMEM((W, HALF), F32)] * DEPTH1
        + [pltpu.SemaphoreType.DMA] * (DEPTH1 + 1)
    ),
    compiler_params=pltpu.CompilerParams(use_tc_tiling_on_sc=False),
)


def _out_body(bs_ref, batch_ref, h1_ref, a1A_ref, a1B_ref, degA_ref,
              degB_ref, r1_ref, bl1_ref, f1w_ref, f1b_ref, f2w_ref,
              f2b_ref, o_ref, acc_ref, hv_ref):
    i = pl.program_id(0)
    deg = jnp.maximum(degA_ref[:, 0:1] + degB_ref[:, 0:1], 1.0)
    h2 = (jnp.concatenate([a1A_ref[...], a1B_ref[...]], axis=1) / deg
          + bl1_ref[...] + r1_ref[...])
    hv_ref[...] = jnp.concatenate([h1_ref[...], h2], axis=1)

    @pl.when(i == 0)
    def _():
        acc_ref[...] = jnp.full((NG, 2 * DH), -jnp.inf, F32)

    bmin = bs_ref[i * RBC]
    bmax = bs_ref[i * RBC + RBC - 1]
    gi = lax.broadcasted_iota(jnp.int32, (NG, 1), 0)

    def body(g, carry):
        m = jnp.where(batch_ref[...] == g, hv_ref[...], -jnp.inf)
        m = jnp.max(m, axis=0, keepdims=True)
        acc_ref[...] = jnp.where(gi == g, jnp.maximum(acc_ref[...], m),
                                 acc_ref[...])
        return carry

    lax.fori_loop(bmin, bmax + 1, body, 0)

    @pl.when(i == pl.num_programs(0) - 1)
    def _():
        z = jnp.maximum(
            jnp.dot(acc_ref[...], f1w_ref[...], preferred_element_type=F32)
            + f1b_ref[...], 0.0)
        o_ref[...] = (jnp.dot(z, f2w_ref[...], preferred_element_type=F32)
                      + f2b_ref[...])


_out_call = pl.pallas_call(
    _out_body,
    grid_spec=pltpu.PrefetchScalarGridSpec(
        num_scalar_prefetch=1,
        grid=(GRIDC,),
        in_specs=[
            pl.BlockSpec((RBC, 1), lambda i, b: (i, 0)),
            pl.BlockSpec((RBC, DH), lambda i, b: (i, 0)),
            pl.BlockSpec((RBC, HALF), lambda i, b: (i, 0)),
            pl.BlockSpec((RBC, HALF), lambda i, b: (i + GRIDC, 0)),
            pl.BlockSpec((RBC, 16), lambda i, b: (i, 0)),
            pl.BlockSpec((RBC, 16), lambda i, b: (i + GRIDC, 0)),
            pl.BlockSpec((RBC, DH), lambda i, b: (i, 0)),
            pl.BlockSpec((1, DH), lambda i, b: (0, 0)),
            pl.BlockSpec((2 * DH, 256), lambda i, b: (0, 0)),
            pl.BlockSpec((1, 256), lambda i, b: (0, 0)),
            pl.BlockSpec((256, 10), lambda i, b: (0, 0)),
            pl.BlockSpec((1, 10), lambda i, b: (0, 0)),
        ],
        out_specs=pl.BlockSpec((NG, 10), lambda i, b: (0, 0)),
        scratch_shapes=[
            pltpu.VMEM((NG, 2 * DH), F32),
            pltpu.VMEM((RBC, 2 * DH), F32),
        ],
    ),
    out_shape=jax.ShapeDtypeStruct((NG, 10), F32),
)


@jax.jit
def _run(x, edge_index, batch, eigen_values, W_l0, b_l0, W_r0,
         W_l1, b_l1, W_r1, fc1_w, fc1_b, fc2_w, fc2_b):
    edges = edge_index.reshape(2 * NROW, W)
    batch2 = batch.reshape(N, 1)
    deg = _deg_call(edges)
    t0 = _t0_call(x, batch2, eigen_values)
    agg0 = _l0_call(t0, edges)
    h1, p1, r1 = _mid_call(t0, agg0, agg0, deg, deg,
                           W_l0, b_l0.reshape(1, DH), W_r0, W_l1, W_r1)
    agg1 = _l1_call(p1[0], p1[1], edges)
    return _out_call(batch, batch2, h1, agg1, agg1, deg, deg, r1,
                     b_l1.reshape(1, DH), fc1_w, fc1_b.reshape(1, 256),
                     fc2_w, fc2_b.reshape(1, 10))


def kernel(x, edge_index, batch, eigen_values, W_l0, b_l0, W_r0,
           W_l1, b_l1, W_r1, fc1_w, fc1_b, fc2_w, fc2_b):
    return _run(x, edge_index, batch, eigen_values, W_l0, b_l0, W_r0,
                W_l1, b_l1, W_r1, fc1_w, fc1_b, fc2_w, fc2_b)


# speedup vs baseline: 1.0314x; 1.0085x over previous
import jax
import jax.numpy as jnp
from jax import lax
from jax.experimental import pallas as pl
from jax.experimental.pallas import tpu as pltpu
from jax.experimental.pallas import tpu_sc as plsc

N = 10000
E = 320000
DF = 128
DI = 32
D0 = DF + DI
DH = 224
HALF = DH // 2
NG = 64
NC = 2
NS = 16
TAB = 10240
ROWS_PT = TAB // NS
LAST_PT = N - (NS - 1) * ROWS_PT
W = 80
NROW = E // W
DEPTH = 2
DEPTH1 = 2
RB = 1000
GRID = N // RB
RBC = 1000
GRIDC = N // RBC
F32 = jnp.float32

_sc_mesh = plsc.VectorSubcoreMesh(core_axis_name="c", subcore_axis_name="s")


def _t0_body(x_ref, b_ref, s_ref, o_ref):
    oh = (b_ref[...] == lax.broadcasted_iota(jnp.int32, (1, NG), 1)).astype(F32)
    rep = jnp.dot(oh, s_ref[...], preferred_element_type=F32)
    o_ref[...] = jnp.concatenate([x_ref[...], rep], axis=1)


_t0_call = pl.pallas_call(
    _t0_body,
    grid=(GRID,),
    in_specs=[
        pl.BlockSpec((RB, DF), lambda i: (i, 0)),
        pl.BlockSpec((RB, 1), lambda i: (i, 0)),
        pl.BlockSpec((NG, DI), lambda i: (0, 0)),
    ],
    out_specs=pl.BlockSpec((RB, D0), lambda i: (i, 0)),
    out_shape=jax.ShapeDtypeStruct((N, D0), F32),
)


def _sc_deg_body(edges_hbm, deg_hbm, deg_sh, ones_v, zd_v, idst_v, sem):
    c = lax.axis_index("c")
    s = lax.axis_index("s")
    wid = c * NS + s
    zeros16 = jnp.zeros((16,), F32)
    ones16 = jnp.ones((16,), F32)

    @pl.loop(0, W)
    def _(i):
        ones_v[i] = ones16
        zd_v[i] = zeros16

    rbase = s * ROWS_PT
    @pl.loop(0, ROWS_PT // W)
    def _(k):
        pltpu.sync_copy(zd_v, deg_sh.at[pl.ds(rbase + k * W, W)])

    nch = NROW // (NC * NS)
    pltpu.sync_copy(edges_hbm.at[pl.ds(NROW + wid * nch, nch)], idst_v)
    plsc.subcore_barrier()

    @pl.loop(0, nch, step=5)
    def _(ch):
        for k in range(5):
            pltpu.async_copy(ones_v, deg_sh.at[idst_v.at[ch + k]], sem,
                             add=True)
        for k in range(5):
            pltpu.make_async_copy(ones_v, deg_sh.at[idst_v.at[ch + k]],
                                  sem).wait()

    plsc.subcore_barrier()

    obase = c * N + rbase

    @pl.when(s < NS - 1)
    def _():
        pltpu.sync_copy(deg_sh.at[pl.ds(rbase, ROWS_PT)],
                        deg_hbm.at[pl.ds(obase, ROWS_PT)])

    @pl.when(s == NS - 1)
    def _():
        pltpu.sync_copy(deg_sh.at[pl.ds(rbase, LAST_PT)],
                        deg_hbm.at[pl.ds(obase, LAST_PT)])


_deg_call = pl.kernel(
    _sc_deg_body,
    out_type=jax.ShapeDtypeStruct((NC * N, 16), F32),
    mesh=_sc_mesh,
    scratch_types=[
        pltpu.VMEM_SHARED((TAB, 16), F32),
        pltpu.VMEM((W, 16), F32),
        pltpu.VMEM((W, 16), F32),
        pltpu.VMEM((NROW // (NC * NS), W), jnp.int32),
        pltpu.SemaphoreType.DMA,
    ],
    compiler_params=pltpu.CompilerParams(use_tc_tiling_on_sc=False),
)


def _edge_pipeline(tbl_hbm, edges_hbm, tab_sh,
                   isrc_v, idst_v, rows_bufs, sems, semi,
                   tile_row_base, n_chunks):
    src_hbm = dst_hbm = edges_hbm
    dst_off = NROW
    depth = len(rows_bufs)
    n_blocks = n_chunks // depth
    tail = n_chunks % depth

    def start(idx_row, rows_v, sem):
        return pltpu.async_copy(tbl_hbm.at[isrc_v.at[idx_row]], rows_v, sem)

    def wait(idx_row, rows_v, sem):
        pltpu.make_async_copy(tbl_hbm.at[isrc_v.at[idx_row]], rows_v,
                              sem).wait()

    def scatter(idx_row, rows_v):
        pltpu.sync_copy(rows_v, tab_sh.at[idst_v.at[idx_row]], add=True)

    def idx_load(blk, half, sync, rows=depth):
        nb = tile_row_base + blk * depth
        if sync:
            pltpu.sync_copy(src_hbm.at[pl.ds(nb, rows)],
                            isrc_v.at[pl.ds(half, rows)])
            pltpu.sync_copy(dst_hbm.at[pl.ds(nb + dst_off, rows)],
                            idst_v.at[pl.ds(half, rows)])
        else:
            pltpu.async_copy(src_hbm.at[pl.ds(nb, rows)],
                             isrc_v.at[pl.ds(half, rows)], semi)
            pltpu.async_copy(dst_hbm.at[pl.ds(nb + dst_off, rows)],
                             idst_v.at[pl.ds(half, rows)], semi)

    def idx_wait(blk, half):
        nb = tile_row_base + blk * depth
        pltpu.make_async_copy(src_hbm.at[pl.ds(nb, depth)],
                              isrc_v.at[pl.ds(half, depth)], semi).wait()
        pltpu.make_async_copy(dst_hbm.at[pl.ds(nb + dst_off, depth)],
                              idst_v.at[pl.ds(half, depth)], semi).wait()

    idx_load(0, 0, True)
    start(0, rows_bufs[0], sems[0])
    if n_blocks > 1:
        idx_load(1, depth, False)

    @pl.loop(0, n_blocks)
    def _(blk):
        half = lax.rem(blk, 2) * depth
        nhalf = depth - half
        for j in range(depth - 1):
            start(half + j + 1, rows_bufs[j + 1], sems[j + 1])
        wait(half, rows_bufs[0], sems[0])
        scatter(half, rows_bufs[0])

        @pl.when(blk < n_blocks - 1)
        def _():
            idx_wait(blk + 1, nhalf)
            start(nhalf, rows_bufs[0], sems[0])

        for j in range(1, depth):
            wait(half + j, rows_bufs[j], sems[j])
            scatter(half + j, rows_bufs[j])

        @pl.when(blk < n_blocks - 2)
        def _():
            idx_load(blk + 2, half, False)

    if tail:
        last = tile_row_base + n_blocks * depth
        pltpu.sync_copy(src_hbm.at[pl.ds(last, tail)],
                        isrc_v.at[pl.ds(0, tail)])
        pltpu.sync_copy(dst_hbm.at[pl.ds(last + dst_off, tail)],
                        idst_v.at[pl.ds(0, tail)])
        for j in range(tail):
            start(j, rows_bufs[0], sems[0]).wait()
            scatter(j, rows_bufs[0])


def _zero_fill(rows_v, tab_sh, s, width):
    zeros16 = jnp.zeros((16,), F32)

    @pl.loop(0, W)
    def _(i):
        @pl.loop(0, width // 16)
        def _(j):
            rows_v[i, pl.ds(j * 16, 16)] = zeros16

    rbase = s * ROWS_PT
    @pl.loop(0, ROWS_PT // W)
    def _(k):
        pltpu.sync_copy(rows_v, tab_sh.at[pl.ds(rbase + k * W, W)])


def _write_out(tab_sh, out_hbm, c, s):
    rbase = s * ROWS_PT
    obase = c * N + rbase

    @pl.when(s < NS - 1)
    def _():
        pltpu.sync_copy(tab_sh.at[pl.ds(rbase, ROWS_PT)],
                        out_hbm.at[pl.ds(obase, ROWS_PT)])

    @pl.when(s == NS - 1)
    def _():
        pltpu.sync_copy(tab_sh.at[pl.ds(rbase, LAST_PT)],
                        out_hbm.at[pl.ds(obase, LAST_PT)])


def _sc_l0_body(t0_hbm, edges_hbm, agg_hbm,
                tab_sh, isrc_v, idst_v, *bufs_sems):
    bufs = list(bufs_sems[:DEPTH])
    sems = list(bufs_sems[DEPTH:2 * DEPTH])
    semi = bufs_sems[2 * DEPTH]
    c = lax.axis_index("c")
    s = lax.axis_index("s")
    wid = c * NS + s
    _zero_fill(bufs[0], tab_sh, s, D0)
    plsc.subcore_barrier()
    _edge_pipeline(t0_hbm, edges_hbm, tab_sh,
                   isrc_v, idst_v, bufs, sems,
                   semi, wid * (NROW // (NC * NS)), NROW // (NC * NS))
    plsc.subcore_barrier()
    _write_out(tab_sh, agg_hbm, c, s)


_l0_call = pl.kernel(
    _sc_l0_body,
    out_type=jax.ShapeDtypeStruct((NC * N, D0), F32),
    mesh=_sc_mesh,
    scratch_types=(
        [pltpu.VMEM_SHARED((TAB, D0), F32),
         pltpu.VMEM((2 * DEPTH, W), jnp.int32),
         pltpu.VMEM((2 * DEPTH, W), jnp.int32)]
        + [pltpu.VMEM((W, D0), F32)] * DEPTH
        + [pltpu.SemaphoreType.DMA] * (DEPTH + 1)
    ),
    compiler_params=pltpu.CompilerParams(use_tc_tiling_on_sc=False),
)


def _mid_body(t0_ref, aggA_ref, aggB_ref, degA_ref, degB_ref,
              wl0_ref, bl0_ref, wr0_ref, wl1_ref, wr1_ref,
              h1_ref, p1_ref, r1_ref):
    deg = jnp.maximum(degA_ref[:, 0:1] + degB_ref[:, 0:1], 1.0)
    mean = (aggA_ref[...] + aggB_ref[...]) / deg
    t0 = t0_ref[...]
    h1 = (jnp.dot(mean, wl0_ref[...], preferred_element_type=F32)
          + bl0_ref[...]
          + jnp.dot(t0, wr0_ref[...], preferred_element_type=F32))
    h1_ref[...] = h1
    rep = t0[:, DF:]
    p1 = (jnp.dot(h1, wl1_ref[0:DH, :], preferred_element_type=F32)
          + jnp.dot(rep, wl1_ref[DH:, :], preferred_element_type=F32))
    p1_ref[0] = p1[:, :HALF]
    p1_ref[1] = p1[:, HALF:]
    r1_ref[...] = (jnp.dot(h1, wr1_ref[0:DH, :], preferred_element_type=F32)
                   + jnp.dot(rep, wr1_ref[DH:, :], preferred_element_type=F32))


_mid_call = pl.pallas_call(
    _mid_body,
    grid=(GRID,),
    in_specs=[
        pl.BlockSpec((RB, D0), lambda i: (i, 0)),
        pl.BlockSpec((RB, D0), lambda i: (i, 0)),
        pl.BlockSpec((RB, D0), lambda i: (i + GRID, 0)),
        pl.BlockSpec((RB, 16), lambda i: (i, 0)),
        pl.BlockSpec((RB, 16), lambda i: (i + GRID, 0)),
        pl.BlockSpec((D0, DH), lambda i: (0, 0)),
        pl.BlockSpec((1, DH), lambda i: (0, 0)),
        pl.BlockSpec((D0, DH), lambda i: (0, 0)),
        pl.BlockSpec((DH + DI, DH), lambda i: (0, 0)),
        pl.BlockSpec((DH + DI, DH), lambda i: (0, 0)),
    ],
    out_specs=[
        pl.BlockSpec((RB, DH), lambda i: (i, 0)),
        pl.BlockSpec((2, RB, HALF), lambda i: (0, i, 0)),
        pl.BlockSpec((RB, DH), lambda i: (i, 0)),
    ],
    out_shape=[
        jax.ShapeDtypeStruct((N, DH), F32),
        jax.ShapeDtypeStruct((2, N, HALF), F32),
        jax.ShapeDtypeStruct((N, DH), F32),
    ],
)


def _sc_l1_body(p1a_hbm, p1b_hbm, edges_hbm, out_hbm,
                tab_sh, isrc_v, idst_v, *bufs_sems):
    bufs = list(bufs_sems[:DEPTH1])
    sems = list(bufs_sems[DEPTH1:2 * DEPTH1])
    semi = bufs_sems[2 * DEPTH1]
    c = lax.axis_index("c")
    s = lax.axis_index("s")
    _zero_fill(bufs[0], tab_sh, s, HALF)
    plsc.subcore_barrier()

    trb = s * (NROW // NS)
    nch = NROW // NS

    @pl.when(c == 0)
    def _():
        _edge_pipeline(p1a_hbm, edges_hbm, tab_sh,
                       isrc_v, idst_v, bufs, sems, semi, trb, nch)

    @pl.when(c == 1)
    def _():
        _edge_pipeline(p1b_hbm, edges_hbm, tab_sh,
                       isrc_v, idst_v, bufs, sems, semi, trb, nch)

    plsc.subcore_barrier()
    _write_out(tab_sh, out_hbm, c, s)


_l1_call = pl.kernel(
    _sc_l1_body,
    out_type=jax.ShapeDtypeStruct((NC * N, HALF), F32),
    mesh=_sc_mesh,
    scratch_types=(
        [pltpu.VMEM_SHARED((TAB, HALF), F32),
         pltpu.VMEM((2 * DEPTH1, W), jnp.int32),
         pltpu.VMEM((2 * DEPTH1, W), jnp.int32)]
        + [pltpu.VMEM((W, HALF), F32)] * DEPTH1
        + [pltpu.SemaphoreType.DMA] * (DEPTH1 + 1)
    ),
    compiler_params=pltpu.CompilerParams(use_tc_tiling_on_sc=False),
)


def _segmax_block(bs_ref, batch_ref, val_fn, acc_ref, i, width):
    @pl.when(i == 0)
    def _():
        acc_ref[...] = jnp.full((NG, width), -jnp.inf, F32)

    bmin = bs_ref[i * RBC]
    bmax = bs_ref[i * RBC + RBC - 1]
    gi = lax.broadcasted_iota(jnp.int32, (NG, 1), 0)

    def body(g, carry):
        m = jnp.where(batch_ref[...] == g, val_fn(), -jnp.inf)
        m = jnp.max(m, axis=0, keepdims=True)
        acc_ref[...] = jnp.where(gi == g, jnp.maximum(acc_ref[...], m),
                                 acc_ref[...])
        return carry

    lax.fori_loop(bmin, bmax + 1, body, 0)


def _pool1_body(bs_ref, batch_ref, h1_ref, o_ref, acc_ref):
    i = pl.program_id(0)
    _segmax_block(bs_ref, batch_ref, lambda: h1_ref[...], acc_ref, i, DH)

    @pl.when(i == pl.num_programs(0) - 1)
    def _():
        o_ref[...] = acc_ref[...]


_pool1_call = pl.pallas_call(
    _pool1_body,
    grid_spec=pltpu.PrefetchScalarGridSpec(
        num_scalar_prefetch=1,
        grid=(GRIDC,),
        in_specs=[
            pl.BlockSpec((RBC, 1), lambda i, b: (i, 0)),
            pl.BlockSpec((RBC, DH), lambda i, b: (i, 0)),
        ],
        out_specs=pl.BlockSpec((NG, DH), lambda i, b: (0, 0)),
        scratch_shapes=[pltpu.VMEM((NG, DH), F32)],
    ),
    out_shape=jax.ShapeDtypeStruct((NG, DH), F32),
)


def _out_body(bs_ref, batch_ref, a1A_ref, a1B_ref, degA_ref,
              degB_ref, r1_ref, bl1_ref, p1_ref, f1w_ref, f1b_ref,
              f2w_ref, f2b_ref, o_ref, acc_ref, hv_ref):
    i = pl.program_id(0)
    deg = jnp.maximum(degA_ref[:, 0:1] + degB_ref[:, 0:1], 1.0)
    hv_ref[...] = (jnp.concatenate([a1A_ref[...], a1B_ref[...]], axis=1)
                   / deg + bl1_ref[...] + r1_ref[...])
    _segmax_block(bs_ref, batch_ref, lambda: hv_ref[...], acc_ref, i, DH)

    @pl.when(i == pl.num_programs(0) - 1)
    def _():
        z = jnp.maximum(
            jnp.dot(p1_ref[...], f1w_ref[0:DH, :],
                    preferred_element_type=F32)
            + jnp.dot(acc_ref[...], f1w_ref[DH:, :],
                      preferred_element_type=F32)
            + f1b_ref[...], 0.0)
        o_ref[...] = (jnp.dot(z, f2w_ref[...], preferred_element_type=F32)
                      + f2b_ref[...])


_out_call = pl.pallas_call(
    _out_body,
    grid_spec=pltpu.PrefetchScalarGridSpec(
        num_scalar_prefetch=1,
        grid=(GRIDC,),
        in_specs=[
            pl.BlockSpec((RBC, 1), lambda i, b: (i, 0)),
            pl.BlockSpec((RBC, HALF), lambda i, b: (i, 0)),
            pl.BlockSpec((RBC, HALF), lambda i, b: (i + GRIDC, 0)),
            pl.BlockSpec((RBC, 16), lambda i, b: (i, 0)),
            pl.BlockSpec((RBC, 16), lambda i, b: (i + GRIDC, 0)),
            pl.BlockSpec((RBC, DH), lambda i, b: (i, 0)),
            pl.BlockSpec((1, DH), lambda i, b: (0, 0)),
            pl.BlockSpec((NG, DH), lambda i, b: (0, 0)),
            pl.BlockSpec((2 * DH, 256), lambda i, b: (0, 0)),
            pl.BlockSpec((1, 256), lambda i, b: (0, 0)),
            pl.BlockSpec((256, 10), lambda i, b: (0, 0)),
            pl.BlockSpec((1, 10), lambda i, b: (0, 0)),
        ],
        out_specs=pl.BlockSpec((NG, 10), lambda i, b: (0, 0)),
        scratch_shapes=[
            pltpu.VMEM((NG, DH), F32),
            pltpu.VMEM((RBC, DH), F32),
        ],
    ),
    out_shape=jax.ShapeDtypeStruct((NG, 10), F32),
)


@jax.jit
def _run(x, edge_index, batch, eigen_values, W_l0, b_l0, W_r0,
         W_l1, b_l1, W_r1, fc1_w, fc1_b, fc2_w, fc2_b):
    edges = edge_index.reshape(2 * NROW, W)
    batch2 = batch.reshape(N, 1)
    deg = _deg_call(edges)
    t0 = _t0_call(x, batch2, eigen_values)
    agg0 = _l0_call(t0, edges)
    h1, p1, r1 = _mid_call(t0, agg0, agg0, deg, deg,
                           W_l0, b_l0.reshape(1, DH), W_r0, W_l1, W_r1)
    pooled1 = _pool1_call(batch, batch2, h1)
    agg1 = _l1_call(p1[0], p1[1], edges)
    return _out_call(batch, batch2, agg1, agg1, deg, deg, r1,
                     b_l1.reshape(1, DH), pooled1, fc1_w,
                     fc1_b.reshape(1, 256), fc2_w, fc2_b.reshape(1, 10))


def kernel(x, edge_index, batch, eigen_values, W_l0, b_l0, W_r0,
           W_l1, b_l1, W_r1, fc1_w, fc1_b, fc2_w, fc2_b):
    return _run(x, edge_index, batch, eigen_values, W_l0, b_l0, W_r0,
                W_l1, b_l1, W_r1, fc1_w, fc1_b, fc2_w, fc2_b)
